# baseline (speedup 1.0000x reference)
import numpy as np

import jax
import jax.numpy as jnp
from jax.experimental import pallas as pl
from jax.experimental.pallas import tpu as pltpu

_K = 5
_H_IN = 64
_H1 = 60
_P1 = 30
_H2 = 26
_P2 = 13
_C1_IN, _C1_OUT = 3, 6
_C2_IN, _C2_OUT = 6, 16
_SLOT = 256
_KW1 = _K * _SLOT
_KW2 = _K * _SLOT
_N1 = _C1_OUT * _H1
_N2 = _C2_OUT * _H2
_NH1 = _N1 // 2
_NH2 = _N2 // 2
_FW = _P2 * _SLOT
_NB = 64


def _band(weights, win, wout, slot=None):
    k, _, ci_n, co_n = weights.shape
    t = jnp.zeros((k, win, 2 * wout, ci_n, co_n), jnp.float32)
    for kw in range(k):
        diag = jnp.eye(win, 2 * wout, -kw, dtype=jnp.float32)
        t = t + diag[None, :, :, None, None] * weights[:, kw][:, None, None]
    t = t.transpose(0, 3, 1, 4, 2)
    even = t[..., 0::2].reshape(k, ci_n, win, co_n * wout)
    odd = t[..., 1::2].reshape(k, ci_n, win, co_n * wout)
    cat = jnp.concatenate([even, odd], axis=-1)
    if slot is None:
        perci = cat.transpose(1, 0, 2, 3)
        return perci.reshape(ci_n, k * win,
                             2 * co_n * wout).astype(jnp.bfloat16)
    rows = cat.reshape(k, ci_n * win, 2 * co_n * wout)
    rows = jnp.pad(rows, ((0, 0), (0, slot - ci_n * win), (0, 0)))
    return rows.reshape(k * slot, 2 * co_n * wout).astype(jnp.bfloat16)


def _fused_kernel(x3t_ref, b1m_ref, b2m_ref, wf1_ref, wf2_ref, wf3_ref,
                  b1r_ref, b2r_ref, bf1_ref, bf2_ref, bf3_ref,
                  o_ref, p1_ref, f_ref):
    f32 = jnp.float32
    bf16 = jnp.bfloat16
    b2m = b2m_ref[...]
    xb_ref = x3t_ref

    for p in range(_P1):
        acc = None
        for ci in range(_C1_IN):
            base = ci * _H_IN * _H_IN
            a0 = xb_ref[:, base + 2 * p * _H_IN:
                        base + (2 * p + _K) * _H_IN]
            a1 = xb_ref[:, base + (2 * p + 1) * _H_IN:
                        base + (2 * p + 1 + _K) * _H_IN]
            a = jnp.concatenate([a0, a1], axis=0)
            d = jnp.dot(a, b1m_ref[ci], preferred_element_type=f32)
            acc = d if acc is None else acc + d
        m = jnp.maximum(acc[:, :_NH1], acc[:, _NH1:])
        v = jnp.maximum(m[:_NB], m[_NB:])
        y = jnp.maximum(v + b1r_ref[...], 0.0)
        row = jnp.concatenate(
            [y.astype(bf16), jnp.zeros((_NB, _SLOT - _NH1), bf16)], axis=1)
        p1_ref[:, p * _SLOT:(p + 1) * _SLOT] = row

    for p in range(_P2):
        h0 = 2 * p
        a0 = p1_ref[:, h0 * _SLOT: h0 * _SLOT + _KW2]
        a1 = p1_ref[:, (h0 + 1) * _SLOT: (h0 + 1) * _SLOT + _KW2]
        a = jnp.concatenate([a0, a1], axis=0)
        o = jnp.dot(a, b2m, preferred_element_type=f32)
        m = jnp.maximum(o[:, :_NH2], o[:, _NH2:])
        v = jnp.maximum(m[:_NB], m[_NB:])
        y = jnp.maximum(v + b2r_ref[...], 0.0)
        row = jnp.concatenate(
            [y.astype(bf16), jnp.zeros((_NB, _SLOT - _NH2), bf16)], axis=1)
        f_ref[:, p * _SLOT:(p + 1) * _SLOT] = row

    feats = f_ref[...]
    h1 = jnp.maximum(
        jnp.dot(feats, wf1_ref[...], preferred_element_type=f32)
        + bf1_ref[...], 0.0)
    h2 = jnp.maximum(
        jnp.dot(h1, wf2_ref[...], preferred_element_type=f32)
        + bf2_ref[...], 0.0)
    logits = (jnp.dot(h2, wf3_ref[...], preferred_element_type=f32)
              + bf3_ref[...])
    m = jnp.max(logits, axis=1, keepdims=True)
    s = logits - m
    lse = jnp.log(jnp.sum(jnp.exp(s), axis=1, keepdims=True))
    o_ref[...] = s - lse


def kernel(x, w1c, b1c, w2c, b2c, wf1, bf1, wf2, bf2, wf3, bf3):
    n = x.shape[0]
    n_pad = ((n + _NB - 1) // _NB) * _NB

    x3t = x.reshape(n, _C1_IN * _H_IN * _H_IN).astype(jnp.bfloat16)
    if n_pad != n:
        x3t = jnp.pad(x3t, ((0, n_pad - n), (0, 0)))

    b1m = _band(w1c, _H_IN, _P1)
    b2m = _band(w2c, _P1, _P2, slot=_SLOT)

    wf1p = wf1.reshape(_C2_OUT, _P2, _P2, wf1.shape[1])
    wf1p = wf1p.transpose(1, 0, 2, 3).reshape(_P2, _NH2, wf1.shape[1])
    wf1p = jnp.pad(wf1p, ((0, 0), (0, _SLOT - _NH2), (0, 0)))
    wf1p = wf1p.reshape(_FW, wf1.shape[1]).astype(jnp.bfloat16)

    b1r = jnp.repeat(b1c, _P1)[None, :]
    b2r = jnp.repeat(b2c, _P2)[None, :]

    out = pl.pallas_call(
        _fused_kernel,
        out_shape=jax.ShapeDtypeStruct((n_pad, 2), jnp.float32),
        grid=(n_pad // _NB,),
        in_specs=[
            pl.BlockSpec((_NB, _C1_IN * _H_IN * _H_IN), lambda i: (i, 0)),
            pl.BlockSpec(b1m.shape, lambda i: (0, 0, 0)),
            pl.BlockSpec(b2m.shape, lambda i: (0, 0)),
            pl.BlockSpec(wf1p.shape, lambda i: (0, 0)),
            pl.BlockSpec(wf2.shape, lambda i: (0, 0)),
            pl.BlockSpec(wf3.shape, lambda i: (0, 0)),
            pl.BlockSpec((1, _NH1), lambda i: (0, 0)),
            pl.BlockSpec((1, _NH2), lambda i: (0, 0)),
            pl.BlockSpec((1, wf1.shape[1]), lambda i: (0, 0)),
            pl.BlockSpec((1, wf2.shape[1]), lambda i: (0, 0)),
            pl.BlockSpec((1, wf3.shape[1]), lambda i: (0, 0)),
        ],
        out_specs=pl.BlockSpec((_NB, 2), lambda i: (i, 0)),
        scratch_shapes=[
            pltpu.VMEM((_NB, _P1 * _SLOT), jnp.bfloat16),
            pltpu.VMEM((_NB, _FW), jnp.bfloat16),
        ],
        compiler_params=pltpu.CompilerParams(
            dimension_semantics=("parallel",)),
    )(x3t, b1m, b2m, wf1p, wf2, wf3, b1r, b2r,
      bf1.reshape(1, -1), bf2.reshape(1, -1), bf3.reshape(1, -1))
    return out[:n]


# speedup vs baseline: 1.0738x; 1.0738x over previous
import numpy as np

import jax
import jax.numpy as jnp
from jax.experimental import pallas as pl
from jax.experimental.pallas import tpu as pltpu

_K = 5
_H_IN = 64
_H1 = 60
_P1 = 30
_H2 = 26
_P2 = 13
_C1_IN, _C1_OUT = 3, 6
_C2_IN, _C2_OUT = 6, 16
_SLOT = 256
_KW1 = _K * _SLOT
_KW2 = _K * _SLOT
_N1 = _C1_OUT * _H1
_N2 = _C2_OUT * _H2
_NH1 = _N1 // 2
_NH2 = _N2 // 2
_FW = _P2 * _SLOT
_NB = 128


def _band(weights, win, wout, slot=None):
    k, _, ci_n, co_n = weights.shape
    t = jnp.zeros((k, win, 2 * wout, ci_n, co_n), jnp.float32)
    for kw in range(k):
        diag = jnp.eye(win, 2 * wout, -kw, dtype=jnp.float32)
        t = t + diag[None, :, :, None, None] * weights[:, kw][:, None, None]
    t = t.transpose(0, 3, 1, 4, 2)
    even = t[..., 0::2].reshape(k, ci_n, win, co_n * wout)
    odd = t[..., 1::2].reshape(k, ci_n, win, co_n * wout)
    cat = jnp.concatenate([even, odd], axis=-1)
    if slot is None:
        perci = cat.transpose(1, 0, 2, 3)
        return perci.reshape(ci_n, k * win,
                             2 * co_n * wout).astype(jnp.bfloat16)
    rows = cat.reshape(k, ci_n * win, 2 * co_n * wout)
    rows = jnp.pad(rows, ((0, 0), (0, slot - ci_n * win), (0, 0)))
    return rows.reshape(k * slot, 2 * co_n * wout).astype(jnp.bfloat16)


def _fused_kernel(x3t_ref, b1m_ref, b2m_ref, wf1_ref, wf2_ref, wf3_ref,
                  b1r_ref, b2r_ref, bf1_ref, bf2_ref, bf3_ref,
                  o_ref, p1_ref, f_ref):
    f32 = jnp.float32
    bf16 = jnp.bfloat16
    b2m = b2m_ref[...]
    xb_ref = x3t_ref

    for p in range(_P1):
        acc = None
        for ci in range(_C1_IN):
            base = ci * _H_IN * _H_IN
            a0 = xb_ref[:, base + 2 * p * _H_IN:
                        base + (2 * p + _K) * _H_IN]
            a1 = xb_ref[:, base + (2 * p + 1) * _H_IN:
                        base + (2 * p + 1 + _K) * _H_IN]
            a = jnp.concatenate([a0, a1], axis=0)
            d = jnp.dot(a, b1m_ref[ci], preferred_element_type=f32)
            acc = d if acc is None else acc + d
        m = jnp.maximum(acc[:, :_NH1], acc[:, _NH1:])
        v = jnp.maximum(m[:_NB], m[_NB:])
        y = jnp.maximum(v + b1r_ref[...], 0.0)
        row = jnp.concatenate(
            [y.astype(bf16), jnp.zeros((_NB, _SLOT - _NH1), bf16)], axis=1)
        p1_ref[:, p * _SLOT:(p + 1) * _SLOT] = row

    for p in range(_P2):
        h0 = 2 * p
        a0 = p1_ref[:, h0 * _SLOT: h0 * _SLOT + _KW2]
        a1 = p1_ref[:, (h0 + 1) * _SLOT: (h0 + 1) * _SLOT + _KW2]
        a = jnp.concatenate([a0, a1], axis=0)
        o = jnp.dot(a, b2m, preferred_element_type=f32)
        m = jnp.maximum(o[:, :_NH2], o[:, _NH2:])
        v = jnp.maximum(m[:_NB], m[_NB:])
        y = jnp.maximum(v + b2r_ref[...], 0.0)
        row = jnp.concatenate(
            [y.astype(bf16), jnp.zeros((_NB, _SLOT - _NH2), bf16)], axis=1)
        f_ref[:, p * _SLOT:(p + 1) * _SLOT] = row

    feats = f_ref[...]
    h1 = jnp.maximum(
        jnp.dot(feats, wf1_ref[...], preferred_element_type=f32)
        + bf1_ref[...], 0.0)
    h2 = jnp.maximum(
        jnp.dot(h1, wf2_ref[...], preferred_element_type=f32)
        + bf2_ref[...], 0.0)
    logits = (jnp.dot(h2, wf3_ref[...], preferred_element_type=f32)
              + bf3_ref[...])
    m = jnp.max(logits, axis=1, keepdims=True)
    s = logits - m
    lse = jnp.log(jnp.sum(jnp.exp(s), axis=1, keepdims=True))
    o_ref[...] = s - lse


def kernel(x, w1c, b1c, w2c, b2c, wf1, bf1, wf2, bf2, wf3, bf3):
    n = x.shape[0]
    n_pad = ((n + _NB - 1) // _NB) * _NB

    x3t = x.reshape(n, _C1_IN * _H_IN * _H_IN).astype(jnp.bfloat16)
    if n_pad != n:
        x3t = jnp.pad(x3t, ((0, n_pad - n), (0, 0)))

    b1m = _band(w1c, _H_IN, _P1)
    b2m = _band(w2c, _P1, _P2, slot=_SLOT)

    wf1p = wf1.reshape(_C2_OUT, _P2, _P2, wf1.shape[1])
    wf1p = wf1p.transpose(1, 0, 2, 3).reshape(_P2, _NH2, wf1.shape[1])
    wf1p = jnp.pad(wf1p, ((0, 0), (0, _SLOT - _NH2), (0, 0)))
    wf1p = wf1p.reshape(_FW, wf1.shape[1]).astype(jnp.bfloat16)

    b1r = jnp.repeat(b1c, _P1)[None, :]
    b2r = jnp.repeat(b2c, _P2)[None, :]

    out = pl.pallas_call(
        _fused_kernel,
        out_shape=jax.ShapeDtypeStruct((n_pad, 2), jnp.float32),
        grid=(n_pad // _NB,),
        in_specs=[
            pl.BlockSpec((_NB, _C1_IN * _H_IN * _H_IN), lambda i: (i, 0)),
            pl.BlockSpec(b1m.shape, lambda i: (0, 0, 0)),
            pl.BlockSpec(b2m.shape, lambda i: (0, 0)),
            pl.BlockSpec(wf1p.shape, lambda i: (0, 0)),
            pl.BlockSpec(wf2.shape, lambda i: (0, 0)),
            pl.BlockSpec(wf3.shape, lambda i: (0, 0)),
            pl.BlockSpec((1, _NH1), lambda i: (0, 0)),
            pl.BlockSpec((1, _NH2), lambda i: (0, 0)),
            pl.BlockSpec((1, wf1.shape[1]), lambda i: (0, 0)),
            pl.BlockSpec((1, wf2.shape[1]), lambda i: (0, 0)),
            pl.BlockSpec((1, wf3.shape[1]), lambda i: (0, 0)),
        ],
        out_specs=pl.BlockSpec((_NB, 2), lambda i: (i, 0)),
        scratch_shapes=[
            pltpu.VMEM((_NB, _P1 * _SLOT), jnp.bfloat16),
            pltpu.VMEM((_NB, _FW), jnp.bfloat16),
        ],
        compiler_params=pltpu.CompilerParams(
            dimension_semantics=("parallel",)),
    )(x3t, b1m, b2m, wf1p, wf2, wf3, b1r, b2r,
      bf1.reshape(1, -1), bf2.reshape(1, -1), bf3.reshape(1, -1))
    return out[:n]
